# baseline (speedup 1.0000x reference)
import jax
import jax.numpy as jnp
from jax import lax
from jax.experimental import pallas as pl
from jax.experimental.pallas import tpu as pltpu
from jax.experimental.pallas import tpu_sc as plsc

OBS = 64
SEQ_LEN = 200
BATCH = 4096
OUT_COLS = SEQ_LEN * OBS + 128
L = 16
CCH = 1280
OCH = CCH // OBS
SPLIT = 6400
NCHUNK = 80


def _make_body(col0, with_skill):
    def body(*args):
        if with_skill:
            (seqp, skill, obj_idx, obj_embed, out,
             idxv, embv, skbuf, buf0, buf1,
             semi0, semi1, semo0, semo1, sems) = args
        else:
            (seqp, obj_idx, obj_embed, out,
             idxv, embv, buf0, buf1,
             semi0, semi1, semo0, semo1) = args
        w = lax.axis_index("s") * 2 + lax.axis_index("c")
        bufs = [buf0, buf1]
        sem_in = [semi0, semi1]
        sem_out = [semo0, semo1]

        pltpu.sync_copy(obj_idx, idxv.at[pl.ds(0, BATCH)])
        pltpu.sync_copy(obj_embed, embv)
        evecs = [embv[pl.ds(L * j, L)] for j in range(OBS // L)]

        b0 = pl.multiple_of(w * 128, 128)
        if with_skill:
            skill_in = pltpu.make_async_copy(
                skill.at[pl.ds(b0, 128), :], skbuf, sems)
            skill_in.start()

        def rg(t):
            r_ = pl.multiple_of((w * 16 + t // 5) * 8, 8)
            g_ = t % 5
            return r_, g_

        def start_in(kb, t):
            r_, g_ = rg(t)
            pltpu.make_async_copy(
                seqp.at[pl.ds(r_, 8),
                        pl.ds(pl.multiple_of(CCH * g_, 128), CCH)],
                bufs[kb], sem_in[kb]).start()

        def wait_in(kb):
            pltpu.make_async_copy(
                seqp.at[pl.ds(0, 8), pl.ds(0, CCH)], bufs[kb],
                sem_in[kb]).wait()

        def start_out(kb, t):
            r_, g_ = rg(t)
            pltpu.make_async_copy(
                bufs[kb],
                out.at[pl.ds(r_, 8),
                       pl.ds(pl.multiple_of(col0 + CCH * g_, 128), CCH)],
                sem_out[kb]).start()

        def wait_out(kb):
            pltpu.make_async_copy(
                bufs[kb],
                out.at[pl.ds(0, 8), pl.ds(0, CCH)], sem_out[kb]).wait()

        def apply_embed(kb, t):
            r_, g_ = rg(t)
            iv = idxv[pl.ds(r_, L)]
            o_lo = col0 // OBS + OCH * g_
            for r in range(8):
                o_b = iv[r]
                c0 = OBS * (o_b - o_lo)
                hit = jnp.logical_and(o_b >= o_lo, o_b < o_lo + OCH)

                @pl.when(hit)
                def _():
                    for j in range(OBS // L):
                        bufs[kb][r, pl.ds(c0 + L * j, L)] = (
                            bufs[kb][r, pl.ds(c0 + L * j, L)] + evecs[j])

        start_in(0, 0)
        start_in(1, 1)

        def loop(i, carry):
            for kb in (0, 1):
                t = 2 * i + kb
                wait_in(kb)
                apply_embed(kb, t)
                start_out(kb, t)
                wait_out(kb)

                @pl.when(t + 2 < NCHUNK)
                def _():
                    start_in(kb, t + 2)
            return carry

        lax.fori_loop(0, NCHUNK // 2, loop, 0)

        if with_skill:
            skill_in.wait()
            pltpu.sync_copy(
                skbuf, out.at[pl.ds(b0, 128), pl.ds(SEQ_LEN * OBS, 128)])

    return body


@jax.jit
def kernel(seq, skill, obj_idx, obj_embed):
    obj_idx = obj_idx.astype(jnp.int32)
    seq2d = seq.reshape(BATCH, SEQ_LEN * OBS)
    seq_a = seq2d[:, :SPLIT]
    seq_b = seq2d[:, SPLIT:]
    mesh = plsc.VectorSubcoreMesh(core_axis_name="c", subcore_axis_name="s")
    dma = pltpu.SemaphoreType.DMA

    out = pl.kernel(
        _make_body(0, True),
        out_type=jax.ShapeDtypeStruct((BATCH, OUT_COLS), jnp.float32),
        mesh=mesh,
        scratch_types=[
            pltpu.VMEM((BATCH + L,), jnp.int32),
            pltpu.VMEM((OBS,), jnp.float32),
            pltpu.VMEM((128, 128), jnp.float32),
            pltpu.VMEM((8, CCH), jnp.float32),
            pltpu.VMEM((8, CCH), jnp.float32),
            dma, dma, dma, dma, dma,
        ],
    )(seq_a, skill, obj_idx, obj_embed)

    ref = jax.new_ref(out)
    pl.kernel(
        _make_body(SPLIT, False),
        out_type=(),
        mesh=mesh,
        scratch_types=[
            pltpu.VMEM((BATCH + L,), jnp.int32),
            pltpu.VMEM((OBS,), jnp.float32),
            pltpu.VMEM((8, CCH), jnp.float32),
            pltpu.VMEM((8, CCH), jnp.float32),
            dma, dma, dma, dma,
        ],
    )(seq_b, obj_idx, obj_embed, ref)
    return ref[...]


# speedup vs baseline: 1.3516x; 1.3516x over previous
import jax
import jax.numpy as jnp
from jax import lax
from jax.experimental import pallas as pl
from jax.experimental.pallas import tpu as pltpu
from jax.experimental.pallas import tpu_sc as plsc

OBS = 64
SEQ_LEN = 200
BATCH = 4096
OUT_COLS = SEQ_LEN * OBS + 128
L = 16
OCH = 40
CCH = OCH * OBS
NCHUNK = 80
NBUF = 3


def _sc_kernel(seq, skill, obj_idx, obj_embed, out,
               idxv, embv, skbuf, buf0, buf1, buf2,
               semi0, semi1, semi2, semo0, semo1, semo2, sems):
    w = lax.axis_index("s") * 2 + lax.axis_index("c")
    bufs = [buf0, buf1, buf2]
    sem_in = [semi0, semi1, semi2]
    sem_out = [semo0, semo1, semo2]

    pltpu.sync_copy(obj_idx, idxv.at[pl.ds(0, BATCH)])
    pltpu.sync_copy(obj_embed, embv)
    evecs = [embv[pl.ds(L * j, L)] for j in range(OBS // L)]

    b0 = pl.multiple_of(w * 128, 128)
    skill_in = pltpu.make_async_copy(
        skill.at[pl.ds(b0, 128), :], skbuf, sems)
    skill_in.start()

    def rg(t):
        r_ = pl.multiple_of((w * 16 + t // 5) * 8, 8)
        g_ = t % 5
        return r_, g_

    def start_in(kb, t):
        r_, g_ = rg(t)
        pltpu.make_async_copy(
            seq.at[pl.ds(r_, 8),
                   pl.ds(pl.multiple_of(CCH * g_, 128), CCH)],
            bufs[kb], sem_in[kb]).start()

    def wait_in(kb):
        pltpu.make_async_copy(
            seq.at[pl.ds(0, 8), pl.ds(0, CCH)], bufs[kb],
            sem_in[kb]).wait()

    def start_out(kb, t):
        r_, g_ = rg(t)
        pltpu.make_async_copy(
            bufs[kb],
            out.at[pl.ds(r_, 8),
                   pl.ds(pl.multiple_of(CCH * g_, 128), CCH)],
            sem_out[kb]).start()

    def wait_out(kb):
        pltpu.make_async_copy(
            bufs[kb],
            out.at[pl.ds(0, 8), pl.ds(0, CCH)], sem_out[kb]).wait()

    def apply_embed(kb, t):
        r_, g_ = rg(t)
        iv = idxv[pl.ds(r_, L)]
        o_lo = OCH * g_
        for r in range(8):
            o_b = iv[r]
            c0 = OBS * (o_b - o_lo)
            hit = jnp.logical_and(o_b >= o_lo, o_b < o_lo + OCH)

            @pl.when(hit)
            def _():
                for j in range(OBS // L):
                    bufs[kb][r, pl.ds(c0 + L * j, L)] = (
                        bufs[kb][r, pl.ds(c0 + L * j, L)] + evecs[j])

    def step(kb, t):
        wait_in(kb)
        apply_embed(kb, t)
        start_out(kb, t)
        wait_out(kb)

        @pl.when(t + NBUF < NCHUNK)
        def _():
            start_in(kb, t + NBUF)

    for kb in range(NBUF):
        start_in(kb, kb)

    def body(i, carry):
        for kb in range(NBUF):
            step(kb, NBUF * i + kb)
        return carry

    niter = NCHUNK // NBUF
    lax.fori_loop(0, niter, body, 0)
    for tail in range(NBUF * niter, NCHUNK):
        step(tail % NBUF, tail)

    skill_in.wait()
    pltpu.sync_copy(
        skbuf, out.at[pl.ds(b0, 128), pl.ds(SEQ_LEN * OBS, 128)])


@jax.jit
def kernel(seq, skill, obj_idx, obj_embed):
    obj_idx = obj_idx.astype(jnp.int32)
    seq = seq.reshape(BATCH, SEQ_LEN * OBS)
    mesh = plsc.VectorSubcoreMesh(core_axis_name="c", subcore_axis_name="s")
    dma = pltpu.SemaphoreType.DMA
    out = pl.kernel(
        _sc_kernel,
        out_type=jax.ShapeDtypeStruct((BATCH, OUT_COLS), jnp.float32),
        mesh=mesh,
        scratch_types=[
            pltpu.VMEM((BATCH + L,), jnp.int32),
            pltpu.VMEM((OBS,), jnp.float32),
            pltpu.VMEM((128, 128), jnp.float32),
            pltpu.VMEM((8, CCH), jnp.float32),
            pltpu.VMEM((8, CCH), jnp.float32),
            pltpu.VMEM((8, CCH), jnp.float32),
            dma, dma, dma, dma, dma, dma, dma,
        ],
    )(seq, skill, obj_idx, obj_embed)
    return out
